# Initial kernel scaffold; baseline (speedup 1.0000x reference)
import jax
import jax.numpy as jnp
from jax.experimental import pallas as pl


def kernel(arg0_1, arg1_1, arg2_1):
    raise NotImplementedError("write your pallas kernel here")


# speedup vs baseline: 2.8066x; 2.8066x over previous
import jax
import jax.numpy as jnp
from jax import lax
from jax.experimental import pallas as pl
from jax.experimental.pallas import tpu as pltpu
from jax.experimental.pallas import tpu_sc as plsc

N_ROWS = 1_000_000
D = 32
N_UPD = 16_384
NC = 2
NS = 16
NW = NC * NS
W = 31_232
SC_COLS = 999_936
XTRA = SC_COLS - NW * W
TAILC = N_ROWS - SC_COLS
UTAB = W + XTRA
L = 16
CB = 2048
NFULL = W // CB
REM = W - NFULL * CB
SB = 64
VROW = 128


def _neg_pad_body(a0t_ref, o_ref):
    x = a0t_ref[...]
    o_ref[:, 0:D] = -jnp.transpose(x)
    o_ref[:, D:VROW] = jnp.zeros((x.shape[1], VROW - D), jnp.float32)


def _neg_pad(a0t):
    blk = 2048
    return pl.pallas_call(
        _neg_pad_body,
        out_shape=jax.ShapeDtypeStruct((N_UPD, VROW), jnp.float32),
        grid=(N_UPD // blk,),
        in_specs=[pl.BlockSpec((D, blk), lambda i: (0, i))],
        out_specs=pl.BlockSpec((blk, VROW), lambda i: (i, 0)),
    )(a0t)


def _tail_patch_body(out_ref, a1_ref, a0t_ref, idx_ref, o_ref):
    del out_ref
    acc = a1_ref[...]
    idxg = idx_ref[...]
    posg = (lax.broadcasted_iota(jnp.int32, (128, 128), 0) * 128
            + lax.broadcasted_iota(jnp.int32, (128, 128), 1))
    cvec = lax.broadcasted_iota(jnp.int32, (1, 128), 1)
    wpv = jnp.full((1, 128), -1, jnp.int32)
    for c in range(TAILC):
        sel = jnp.where(idxg == SC_COLS + c, posg, -1)
        wp = jnp.max(sel)
        wpv = jnp.where(cvec == c, wp, wpv)
    valid = wpv >= 0
    sel_mat = (lax.broadcasted_iota(jnp.int32, (N_UPD, 128), 0)
               == jnp.broadcast_to(wpv, (N_UPD, 128))).astype(jnp.float32)
    vals = lax.dot_general(a0t_ref[...], sel_mat, (((1,), (0,)), ((), ())),
                           preferred_element_type=jnp.float32)
    o_ref[...] = jnp.where(jnp.broadcast_to(valid, (D, 128)), -vals, acc)


def _tail_patch(out_t, a1t, a0t, idx):
    idxg = jnp.reshape(idx, (128, 128))
    tb = SC_COLS // 128
    return pl.pallas_call(
        _tail_patch_body,
        out_shape=jax.ShapeDtypeStruct((D, N_ROWS), jnp.float32),
        grid=(1,),
        in_specs=[
            pl.BlockSpec(memory_space=pl.ANY),
            pl.BlockSpec((D, 128), lambda i: (0, tb)),
            pl.BlockSpec((D, N_UPD), lambda i: (0, 0)),
            pl.BlockSpec((128, 128), lambda i: (0, 0)),
        ],
        out_specs=pl.BlockSpec((D, 128), lambda i: (0, tb)),
        input_output_aliases={0: 0},
    )(out_t, a1t, a0t, idxg)


def _merge_chunk(a1t_hbm, out_hbm, neg0p_hbm, utab, buf, clist_pos,
                 clist_col, vals, gat_sem, lo, base, cbc):
    c0 = pl.multiple_of(lo + base, 128)
    pltpu.sync_copy(a1t_hbm.at[:, pl.ds(c0, cbc)], buf.at[:, pl.ds(0, cbc)])

    lane = lax.iota(jnp.int32, L)

    def scan_body(g, ucnt):
        wp = utab[pl.ds(base + g * L, L)]
        m = wp >= 0
        mi = m.astype(jnp.int32)
        pref = plsc.cumsum(mi)
        t = ucnt + pref - 1
        trow = lax.shift_right_logical(jnp.maximum(t, 0), 6)
        tcol = jnp.bitwise_and(t, SB - 1)
        plsc.store_scatter(clist_pos, [trow, tcol], wp, mask=m)
        plsc.store_scatter(clist_col, [trow, tcol], g * L + lane, mask=m)
        return ucnt + pref[L - 1]

    ucnt = lax.fori_loop(0, cbc // L, scan_body, jnp.int32(0), unroll=False)

    def batch_body(b, _):
        pltpu.async_copy(neg0p_hbm.at[clist_pos.at[b]], vals, gat_sem).wait()

        def grp_body(g2, _):
            j = g2 * L + lane
            valid = (b * SB + j) < ucnt
            bvec = jnp.full((L,), b, dtype=jnp.int32)
            ccol = plsc.load_gather(clist_col, [bvec, j], mask=valid)
            for r in range(D):
                rvec = jnp.full((L,), r, dtype=jnp.int32)
                x = plsc.load_gather(vals, [j, rvec], mask=valid)
                plsc.store_scatter(buf, [rvec, ccol], x, mask=valid)
            return 0

        lax.fori_loop(0, SB // L, grp_body, 0, unroll=False)
        return 0

    nb = lax.div(ucnt + SB - 1, jnp.int32(SB))
    lax.fori_loop(0, nb, batch_body, 0, unroll=False)

    pltpu.sync_copy(buf.at[:, pl.ds(0, cbc)], out_hbm.at[:, pl.ds(c0, cbc)])


def _sc_merge_kernel(neg0p_hbm, a1t_hbm, idx_hbm, out_hbm,
                     idx_v, utab, buf, clist_pos, clist_col, vals, gat_sem):
    wid = lax.axis_index("s") * NC + lax.axis_index("c")
    lo = pl.multiple_of(wid * W, 128)
    is_last = wid == NW - 1
    hi = jnp.where(is_last, SC_COLS, lo + W)

    pltpu.sync_copy(idx_hbm, idx_v)

    lane = lax.iota(jnp.int32, L)
    neg1 = jnp.full((L,), -1, dtype=jnp.int32)

    def init_body(i, _):
        utab[pl.ds(i * L, L)] = neg1
        return 0

    lax.fori_loop(0, UTAB // L, init_body, 0, unroll=False)

    def filt_body(g, _):
        v = idx_v[pl.ds(g * L, L)]
        m = (v >= lo) & (v < hi)
        pos = g * L + lane
        plsc.store_scatter(utab, [v - lo], pos, mask=m)
        return 0

    lax.fori_loop(0, N_UPD // L, filt_body, 0, unroll=False)

    zero = jnp.zeros((L,), jnp.int32)

    def seed_body(i, _):
        r = lax.shift_right_logical(i, 2)
        c = jnp.bitwise_and(i, 3) * L + lane
        plsc.store_scatter(clist_pos, [jnp.full((L,), r, jnp.int32), c], zero)
        return 0

    lax.fori_loop(0, (CB // SB) * (SB // L), seed_body, 0, unroll=False)

    args = (a1t_hbm, out_hbm, neg0p_hbm, utab, buf, clist_pos, clist_col,
            vals, gat_sem, lo)

    def chunk_body(k, _):
        _merge_chunk(*args, k * CB, CB)
        return 0

    lax.fori_loop(0, NFULL, chunk_body, 0, unroll=False)
    _merge_chunk(*args, NFULL * CB, REM)

    @pl.when(is_last)
    def _extra():
        _merge_chunk(*args, W, XTRA)


@jax.jit
def _scatter_overwrite(arg0, arg1, idx):
    a0t = jnp.transpose(arg0)
    a1t = jnp.transpose(arg1)
    neg0p = _neg_pad(a0t)
    mesh = plsc.VectorSubcoreMesh(
        core_axis_name="c", subcore_axis_name="s",
        num_cores=NC, num_subcores=NS)
    f = pl.kernel(
        _sc_merge_kernel,
        out_type=jax.ShapeDtypeStruct((D, N_ROWS), jnp.float32),
        mesh=mesh,
        compiler_params=pltpu.CompilerParams(needs_layout_passes=False),
        scratch_types=[
            pltpu.VMEM((N_UPD,), jnp.int32),
            pltpu.VMEM((UTAB,), jnp.int32),
            pltpu.VMEM((D, CB), jnp.float32),
            pltpu.VMEM((CB // SB, SB), jnp.int32),
            pltpu.VMEM((CB // SB, SB), jnp.int32),
            pltpu.VMEM((SB, VROW), jnp.float32),
            pltpu.SemaphoreType.DMA,
        ],
    )
    out_t = f(neg0p, a1t, idx)
    out_t = _tail_patch(out_t, a1t, a0t, idx)
    return jnp.transpose(out_t)


def kernel(arg0_1, arg1_1, arg2_1):
    idx = arg2_1.astype(jnp.int32)
    return (_scatter_overwrite(arg0_1, arg1_1, idx),)
